# baseline (speedup 1.0000x reference)
import dataclasses
import functools

import jax
import jax.numpy as jnp
from jax import lax
from jax.experimental import pallas as pl
from jax.experimental.pallas import tpu as pltpu
from jax.experimental.pallas import tpu_sc as plsc

_N = 10000
_E = 320000
_D = 128
_L = 16
_CH = 80
_NSUB = 16
_NW = 32
_NCH = _E // _CH // _NW
_ZR = _N // _NSUB


def _sc_message_pass(x, meta, edge_attr, wrw, wrb16, zrows):
    mesh = plsc.VectorSubcoreMesh(core_axis_name="core",
                                  subcore_axis_name="subcore")
    cp = pltpu.CompilerParams()
    if "needs_layout_passes" in pltpu.CompilerParams.__dataclass_fields__:
        cp = dataclasses.replace(cp, needs_layout_passes=False)

    big = pltpu.VMEM((_CH, _D), jnp.float32)
    meta_t = pltpu.VMEM((3, _CH), jnp.int32)
    wr_t = pltpu.VMEM((1, _CH), jnp.float32)

    @functools.partial(
        pl.kernel,
        out_type=[
            jax.ShapeDtypeStruct((2, _N, _D), jnp.float32),
            jax.ShapeDtypeStruct((_E // _CH, 1, _CH), jnp.float32),
        ],
        mesh=mesh,
        compiler_params=cp,
        scratch_types=[
            pltpu.VMEM_SHARED((_N, _D), jnp.float32),
            meta_t, meta_t, meta_t, meta_t,
            big, big,
            big, big,
            wr_t, wr_t,
            pltpu.VMEM((_D,), jnp.float32),
            pltpu.VMEM((_L,), jnp.float32),
        ] + [pltpu.SemaphoreType.DMA] * 10,
    )
    def sc_kernel(x_hbm, meta_hbm, ea_hbm, wrw_hbm, wrb_hbm, z_hbm,
                  part_hbm, wrel_hbm,
                  aggr_sh, meta_0, meta_1, meta_2, meta_3,
                  ea_a, ea_b, xj_a, xj_b,
                  wr_a, wr_b, wcoef_v, bias_v,
                  s_idx_a, s_idx_b, s_gat_a, s_gat_b, s_ea_a, s_ea_b,
                  s_sca_a, s_sca_b, s_wr_a, s_wr_b):
        cid = lax.axis_index("core")
        sid = lax.axis_index("subcore")
        wid = sid * 2 + cid
        chunk0 = wid * _NCH

        pltpu.sync_copy(wrw_hbm, wcoef_v)
        pltpu.sync_copy(wrb_hbm, bias_v)
        pltpu.sync_copy(z_hbm, aggr_sh.at[pl.ds(sid * _ZR, _ZR), :])
        plsc.subcore_barrier()

        lanes = lax.iota(jnp.int32, _L)
        zeros_i = jnp.zeros((_L,), jnp.int32)
        lane0 = lanes == 0
        two_i = zeros_i + 2

        def issue_idx(c, meta_v, sem):
            pltpu.async_copy(meta_hbm.at[chunk0 + c], meta_v, sem)

        def wait_idx(meta_v, sem):
            pltpu.make_async_copy(meta_hbm.at[0], meta_v, sem).wait()

        def issue_in(c, meta_v, ea_v, xj_v, s_gat, s_ea):
            pltpu.async_copy(x_hbm.at[meta_v.at[0]], xj_v, s_gat)
            pltpu.async_copy(ea_hbm.at[pl.ds((chunk0 + c) * _CH, _CH), :],
                             ea_v, s_ea)

        def wait_in(ea_v, xj_v, s_gat, s_ea):
            pltpu.make_async_copy(ea_hbm.at[pl.ds(0, _CH), :], xj_v,
                                  s_gat).wait()
            pltpu.make_async_copy(ea_hbm.at[pl.ds(0, _CH), :], ea_v,
                                  s_ea).wait()

        def wait_sca(xj_v, s_sca):
            pltpu.make_async_copy(xj_v, aggr_sh.at[pl.ds(0, _CH), :],
                                  s_sca).wait()

        def wait_wr(wr_v, s_wr):
            pltpu.make_async_copy(wr_v, wrel_hbm.at[0], s_wr).wait()

        def compute(meta_v, ea_v, xj_v, wr_v):
            wv = [wcoef_v[pl.ds(16 * k, _L)] for k in range(_D // _L)]
            bv = bias_v[...]

            @plsc.parallel_loop(0, _CH, 1, unroll=8)
            def _(e):
                if True:
                    ef = zeros_i + e
                    ea_rows = [ea_v[e, pl.ds(16 * k, _L)]
                               for k in range(_D // _L)]
                    prod = [ea_rows[k] * wv[k] for k in range(_D // _L)]
                    acc = ((prod[0] + prod[1]) + (prod[2] + prod[3])) + \
                          ((prod[4] + prod[5]) + (prod[6] + prod[7]))
                    wrel = jnp.full((_L,), jnp.sum(acc), jnp.float32) + bv
                    attn_e = plsc.bitcast(
                        plsc.load_gather(meta_v, [two_i, ef]), jnp.float32)
                    for k in range(_D // _L):
                        xj = xj_v[e, pl.ds(16 * k, _L)]
                        m = jnp.maximum(xj * attn_e + wrel * ea_rows[k], 0.0)
                        xj_v[e, pl.ds(16 * k, _L)] = m
                    plsc.store_scatter(wr_v, [zeros_i, ef], wrel, mask=lane0)

        def issue_out(c, meta_v, xj_v, wr_v, s_sca, s_wr):
            pltpu.async_copy(xj_v, aggr_sh.at[meta_v.at[1]], s_sca, add=True)
            pltpu.async_copy(wr_v, wrel_hbm.at[chunk0 + c], s_wr)

        metas = [meta_0, meta_1, meta_2, meta_3]
        xjs = [xj_a, xj_b]
        eas = [ea_a, ea_b]
        wrs = [wr_a, wr_b]
        s_idx = [s_idx_a, s_idx_b]
        s_gat = [s_gat_a, s_gat_b]
        s_ea = [s_ea_a, s_ea_b]
        s_sca = [s_sca_a, s_sca_b]
        s_wr = [s_wr_a, s_wr_b]
        n_t = (_NCH - 1) // 4

        def position(t, i):
            c = 4 * t + i
            p = i % 2
            q = (i + 1) % 2

            def free_next():
                wait_sca(xjs[q], s_sca[q])
                wait_idx(metas[(i + 1) % 4], s_idx[q])
            if i == 0:
                pl.when(t >= 1)(free_next)
            else:
                free_next()

            issue_in(c + 1, metas[(i + 1) % 4], eas[q], xjs[q],
                     s_gat[q], s_ea[q])

            def prefetch_idx():
                issue_idx(c + 2, metas[(i + 2) % 4], s_idx[p])
            if i == 3:
                pl.when(t < n_t - 1)(prefetch_idx)
            else:
                prefetch_idx()

            wait_in(eas[p], xjs[p], s_gat[p], s_ea[p])

            def free_wr():
                wait_wr(wrs[p], s_wr[p])
            if i in (0, 1):
                pl.when(t >= 1)(free_wr)
            else:
                free_wr()

            compute(metas[i], eas[p], xjs[p], wrs[p])
            issue_out(c, metas[i], xjs[p], wrs[p], s_sca[p], s_wr[p])

        issue_idx(0, meta_0, s_idx_a)
        issue_idx(1, meta_1, s_idx_a)
        wait_idx(meta_0, s_idx_a)
        wait_idx(meta_1, s_idx_a)
        issue_in(0, meta_0, ea_a, xj_a, s_gat_a, s_ea_a)

        @pl.loop(0, n_t)
        def _(t):
            for i in range(4):
                position(t, i)

        wait_sca(xj_b, s_sca_b)
        wait_in(ea_a, xj_a, s_gat_a, s_ea_a)
        wait_wr(wr_a, s_wr_a)
        compute(meta_0, ea_a, xj_a, wr_a)
        issue_out(_NCH - 1, meta_0, xj_a, wr_a, s_sca_a, s_wr_a)
        wait_sca(xj_a, s_sca_a)
        wait_wr(wr_b, s_wr_b)
        wait_wr(wr_a, s_wr_a)

        plsc.subcore_barrier()

        @pl.when(sid == 0)
        def _():
            pltpu.sync_copy(aggr_sh, part_hbm.at[cid])

    return sc_kernel(x, meta, edge_attr, wrw, wrb16, zrows)


def _tc_finish(p0, p1, x, wt, b2d):
    bn = 1000

    def body(p0_ref, p1_ref, x_ref, wt_ref, b_ref, o_ref):
        s = p0_ref[...] + p1_ref[...] + x_ref[...]
        o_ref[...] = (
            jnp.dot(s, wt_ref[...], preferred_element_type=jnp.float32)
            + b_ref[...]
        )

    return pl.pallas_call(
        body,
        grid=(_N // bn,),
        in_specs=[
            pl.BlockSpec((bn, _D), lambda i: (i, 0)),
            pl.BlockSpec((bn, _D), lambda i: (i, 0)),
            pl.BlockSpec((bn, _D), lambda i: (i, 0)),
            pl.BlockSpec((_D, _D), lambda i: (0, 0)),
            pl.BlockSpec((1, _D), lambda i: (0, 0)),
        ],
        out_specs=pl.BlockSpec((bn, _D), lambda i: (i, 0)),
        out_shape=jax.ShapeDtypeStruct((_N, _D), jnp.float32),
    )(p0, p1, x, wt, b2d)


def kernel(x, edge_index, edge_attr, attn, W_R_w, W_R_b, W_nn, b_nn):
    nchunks = _E // _CH
    attn_i = lax.bitcast_convert_type(attn.reshape(_E), jnp.int32)
    meta = jnp.stack(
        [edge_index[0].reshape(nchunks, _CH),
         edge_index[1].reshape(nchunks, _CH),
         attn_i.reshape(nchunks, _CH)], axis=1)
    wrw = W_R_w.reshape(_D)
    wrb16 = jnp.broadcast_to(W_R_b, (_L,))
    zrows = jnp.zeros((_ZR, _D), jnp.float32)

    part, wrel = _sc_message_pass(x, meta, edge_attr, wrw, wrb16, zrows)
    out = _tc_finish(part[0], part[1], x, W_nn.T, b_nn.reshape(1, _D))
    return (out, wrel.reshape(_E, 1))


# speedup vs baseline: 1.0605x; 1.0605x over previous
import dataclasses
import functools

import jax
import jax.numpy as jnp
from jax import lax
from jax.experimental import pallas as pl
from jax.experimental.pallas import tpu as pltpu
from jax.experimental.pallas import tpu_sc as plsc

_N = 10000
_E = 320000
_D = 128
_L = 16
_CH = 80
_NSUB = 16
_NW = 32
_NCH = _E // _CH // _NW
_ZR = _N // _NSUB


def _sc_message_pass(x, meta, edge_attr, wrw, wrb16, zrows):
    mesh = plsc.VectorSubcoreMesh(core_axis_name="core",
                                  subcore_axis_name="subcore")
    cp = pltpu.CompilerParams()
    if "needs_layout_passes" in pltpu.CompilerParams.__dataclass_fields__:
        cp = dataclasses.replace(cp, needs_layout_passes=False)

    big = pltpu.VMEM((_CH, _D), jnp.float32)
    meta_t = pltpu.VMEM((3, _CH), jnp.int32)
    wr_t = pltpu.VMEM((1, _CH), jnp.float32)

    @functools.partial(
        pl.kernel,
        out_type=[
            jax.ShapeDtypeStruct((2, _N, _D), jnp.float32),
            jax.ShapeDtypeStruct((_E // _CH, 1, _CH), jnp.float32),
        ],
        mesh=mesh,
        compiler_params=cp,
        scratch_types=[
            pltpu.VMEM_SHARED((_N, _D), jnp.float32),
            meta_t, meta_t, meta_t, meta_t,
            big, big,
            big, big,
            wr_t, wr_t,
            pltpu.VMEM((_D,), jnp.float32),
            pltpu.VMEM((_L,), jnp.float32),
        ] + [pltpu.SemaphoreType.DMA] * 10,
    )
    def sc_kernel(x_hbm, meta_hbm, ea_hbm, wrw_hbm, wrb_hbm, z_hbm,
                  part_hbm, wrel_hbm,
                  aggr_sh, meta_0, meta_1, meta_2, meta_3,
                  ea_a, ea_b, xj_a, xj_b,
                  wr_a, wr_b, wcoef_v, bias_v,
                  s_idx_a, s_idx_b, s_gat_a, s_gat_b, s_ea_a, s_ea_b,
                  s_sca_a, s_sca_b, s_wr_a, s_wr_b):
        cid = lax.axis_index("core")
        sid = lax.axis_index("subcore")
        wid = sid * 2 + cid
        chunk0 = wid * _NCH

        pltpu.sync_copy(wrw_hbm, wcoef_v)
        pltpu.sync_copy(wrb_hbm, bias_v)
        pltpu.sync_copy(z_hbm, aggr_sh.at[pl.ds(sid * _ZR, _ZR), :])
        plsc.subcore_barrier()

        lanes = lax.iota(jnp.int32, _L)
        zeros_i = jnp.zeros((_L,), jnp.int32)
        lane0 = lanes == 0
        two_i = zeros_i + 2

        def issue_idx(c, meta_v, sem):
            pltpu.async_copy(meta_hbm.at[chunk0 + c], meta_v, sem)

        def wait_idx(meta_v, sem):
            pltpu.make_async_copy(meta_hbm.at[0], meta_v, sem).wait()

        def issue_in(c, meta_v, ea_v, xj_v, s_gat, s_ea):
            pltpu.async_copy(x_hbm.at[meta_v.at[0]], xj_v, s_gat)
            pltpu.async_copy(ea_hbm.at[pl.ds((chunk0 + c) * _CH, _CH), :],
                             ea_v, s_ea)

        def wait_in(ea_v, xj_v, s_gat, s_ea):
            pltpu.make_async_copy(ea_hbm.at[pl.ds(0, _CH), :], xj_v,
                                  s_gat).wait()
            pltpu.make_async_copy(ea_hbm.at[pl.ds(0, _CH), :], ea_v,
                                  s_ea).wait()

        def wait_sca(xj_v, s_sca):
            pltpu.make_async_copy(xj_v, aggr_sh.at[pl.ds(0, _CH), :],
                                  s_sca).wait()

        def wait_wr(wr_v, s_wr):
            pltpu.make_async_copy(wr_v, wrel_hbm.at[0], s_wr).wait()

        def compute(meta_v, ea_v, xj_v, wr_v):
            wv = [wcoef_v[pl.ds(16 * k, _L)] for k in range(_D // _L)]
            bv = bias_v[...]

            @plsc.parallel_loop(0, _CH, 1, unroll=2)
            def _(e):
                if True:
                    ef = zeros_i + e
                    ea_rows = [ea_v[e, pl.ds(16 * k, _L)]
                               for k in range(_D // _L)]
                    prod = [ea_rows[k] * wv[k] for k in range(_D // _L)]
                    acc = ((prod[0] + prod[1]) + (prod[2] + prod[3])) + \
                          ((prod[4] + prod[5]) + (prod[6] + prod[7]))
                    wrel = jnp.full((_L,), jnp.sum(acc), jnp.float32) + bv
                    attn_e = plsc.bitcast(
                        plsc.load_gather(meta_v, [two_i, ef]), jnp.float32)
                    for k in range(_D // _L):
                        xj = xj_v[e, pl.ds(16 * k, _L)]
                        m = jnp.maximum(xj * attn_e + wrel * ea_rows[k], 0.0)
                        xj_v[e, pl.ds(16 * k, _L)] = m
                    plsc.store_scatter(wr_v, [zeros_i, ef], wrel, mask=lane0)

        def issue_out(c, meta_v, xj_v, wr_v, s_sca, s_wr):
            pltpu.async_copy(xj_v, aggr_sh.at[meta_v.at[1]], s_sca, add=True)
            pltpu.async_copy(wr_v, wrel_hbm.at[chunk0 + c], s_wr)

        metas = [meta_0, meta_1, meta_2, meta_3]
        xjs = [xj_a, xj_b]
        eas = [ea_a, ea_b]
        wrs = [wr_a, wr_b]
        s_idx = [s_idx_a, s_idx_b]
        s_gat = [s_gat_a, s_gat_b]
        s_ea = [s_ea_a, s_ea_b]
        s_sca = [s_sca_a, s_sca_b]
        s_wr = [s_wr_a, s_wr_b]
        n_t = (_NCH - 1) // 4

        def position(t, i):
            c = 4 * t + i
            p = i % 2
            q = (i + 1) % 2

            def free_next():
                wait_sca(xjs[q], s_sca[q])
                wait_idx(metas[(i + 1) % 4], s_idx[q])
            if i == 0:
                pl.when(t >= 1)(free_next)
            else:
                free_next()

            issue_in(c + 1, metas[(i + 1) % 4], eas[q], xjs[q],
                     s_gat[q], s_ea[q])

            def prefetch_idx():
                issue_idx(c + 2, metas[(i + 2) % 4], s_idx[p])
            if i == 3:
                pl.when(t < n_t - 1)(prefetch_idx)
            else:
                prefetch_idx()

            wait_in(eas[p], xjs[p], s_gat[p], s_ea[p])

            def free_wr():
                wait_wr(wrs[p], s_wr[p])
            if i in (0, 1):
                pl.when(t >= 1)(free_wr)
            else:
                free_wr()

            compute(metas[i], eas[p], xjs[p], wrs[p])
            issue_out(c, metas[i], xjs[p], wrs[p], s_sca[p], s_wr[p])

        issue_idx(0, meta_0, s_idx_a)
        issue_idx(1, meta_1, s_idx_a)
        wait_idx(meta_0, s_idx_a)
        wait_idx(meta_1, s_idx_a)
        issue_in(0, meta_0, ea_a, xj_a, s_gat_a, s_ea_a)

        @pl.loop(0, n_t)
        def _(t):
            for i in range(4):
                position(t, i)

        wait_sca(xj_b, s_sca_b)
        wait_in(ea_a, xj_a, s_gat_a, s_ea_a)
        wait_wr(wr_a, s_wr_a)
        compute(meta_0, ea_a, xj_a, wr_a)
        issue_out(_NCH - 1, meta_0, xj_a, wr_a, s_sca_a, s_wr_a)
        wait_sca(xj_a, s_sca_a)
        wait_wr(wr_b, s_wr_b)
        wait_wr(wr_a, s_wr_a)

        plsc.subcore_barrier()

        @pl.when(sid == 0)
        def _():
            pltpu.sync_copy(aggr_sh, part_hbm.at[cid])

    return sc_kernel(x, meta, edge_attr, wrw, wrb16, zrows)


def _tc_finish(p0, p1, x, wt, b2d):
    bn = 1000

    def body(p0_ref, p1_ref, x_ref, wt_ref, b_ref, o_ref):
        s = p0_ref[...] + p1_ref[...] + x_ref[...]
        o_ref[...] = (
            jnp.dot(s, wt_ref[...], preferred_element_type=jnp.float32)
            + b_ref[...]
        )

    return pl.pallas_call(
        body,
        grid=(_N // bn,),
        in_specs=[
            pl.BlockSpec((bn, _D), lambda i: (i, 0)),
            pl.BlockSpec((bn, _D), lambda i: (i, 0)),
            pl.BlockSpec((bn, _D), lambda i: (i, 0)),
            pl.BlockSpec((_D, _D), lambda i: (0, 0)),
            pl.BlockSpec((1, _D), lambda i: (0, 0)),
        ],
        out_specs=pl.BlockSpec((bn, _D), lambda i: (i, 0)),
        out_shape=jax.ShapeDtypeStruct((_N, _D), jnp.float32),
    )(p0, p1, x, wt, b2d)


def kernel(x, edge_index, edge_attr, attn, W_R_w, W_R_b, W_nn, b_nn):
    nchunks = _E // _CH
    attn_i = lax.bitcast_convert_type(attn.reshape(_E), jnp.int32)
    meta = jnp.stack(
        [edge_index[0].reshape(nchunks, _CH),
         edge_index[1].reshape(nchunks, _CH),
         attn_i.reshape(nchunks, _CH)], axis=1)
    wrw = W_R_w.reshape(_D)
    wrb16 = jnp.broadcast_to(W_R_b, (_L,))
    zrows = jnp.zeros((_ZR, _D), jnp.float32)

    part, wrel = _sc_message_pass(x, meta, edge_attr, wrw, wrb16, zrows)
    out = _tc_finish(part[0], part[1], x, W_nn.T, b_nn.reshape(1, _D))
    return (out, wrel.reshape(_E, 1))
